# baseline (speedup 1.0000x reference)
import functools

import jax
import jax.numpy as jnp
from jax import lax
from jax.experimental import pallas as pl
from jax.experimental.pallas import tpu as pltpu
from jax.experimental.pallas import tpu_sc as plsc

N = 10000
E = 320000
D = 128
H = 128
C = 10

NC = 2
NS = 16
NW = NC * NS
EW = E // NW
K = 40
NCHUNK = EW // K
NPAD = 10112
RPT = NPAD // NS
NBUF = 3


def _tc_prep(x_ref, we_ref, be_ref, tsrc_ref, tdst_ref):
    x = x_ref[...]
    a = jnp.dot(x, we_ref[:D, :], preferred_element_type=jnp.float32)
    b = jnp.dot(x, we_ref[D:, :], preferred_element_type=jnp.float32) + be_ref[...]
    tsrc_ref[:, :D] = x
    tsrc_ref[:, D:] = a
    tdst_ref[...] = b


def _sc_edge_body(tsrc_hbm, tdst_hbm, src_hbm, dst_hbm, zero_hbm, out_hbm,
                  sidx, didx, sbufs, dbufs, acc, isems, gsems, ssems):
    c = lax.axis_index("c")
    s = lax.axis_index("s")
    w = s * NC + c
    base_e = w * EW
    pltpu.sync_copy(zero_hbm.at[pl.ds(s * RPT, RPT)], acc.at[pl.ds(s * RPT, RPT)])
    plsc.subcore_barrier()

    def fire_idx(ci, b):
        pltpu.async_copy(src_hbm.at[pl.ds(base_e + ci * K, K)], sidx[b], isems[b])
        pltpu.async_copy(dst_hbm.at[pl.ds(base_e + ci * K, K)], didx[b], isems[b])

    def wait_idx(b):
        pltpu.make_async_copy(src_hbm.at[pl.ds(0, K)], sidx[b], isems[b]).wait()
        pltpu.make_async_copy(dst_hbm.at[pl.ds(0, K)], didx[b], isems[b]).wait()

    def fire_gather(b):
        pltpu.async_copy(tsrc_hbm.at[sidx[b]], sbufs[b], gsems[b])
        pltpu.async_copy(tdst_hbm.at[didx[b]], dbufs[b], gsems[b])

    def wait_gather(b):
        pltpu.make_async_copy(tsrc_hbm.at[sidx[b]], sbufs[b], gsems[b]).wait()
        pltpu.make_async_copy(tdst_hbm.at[didx[b]], dbufs[b], gsems[b]).wait()

    def fire_scatter(b):
        pltpu.async_copy(dbufs[b], acc.at[didx[b]], ssems[b], add=True)

    def wait_scatter(b):
        pltpu.make_async_copy(dbufs[b], acc.at[didx[b]], ssems[b]).wait()

    def compute(b):
        srows, drows = sbufs[b], dbufs[b]

        @plsc.parallel_loop(0, K, 1, unroll=8)
        def _(k):
            for j in range(H // 16):
                xv = srows[k, pl.ds(j * 16, 16)]
                av = srows[k, pl.ds(D + j * 16, 16)]
                bv = drows[k, pl.ds(j * 16, 16)]
                drows[k, pl.ds(j * 16, 16)] = xv * jnp.maximum(av + bv, 0.0)

    def chunk_step(ci, b):
        nb = (b + 2) % NBUF
        b1 = (b + 1) % NBUF

        @pl.when(ci + 1 < NCHUNK)
        def _():
            wait_idx(b1)
            fire_gather(b1)

        @pl.when(ci >= 1)
        def _():
            wait_scatter(nb)

        @pl.when(ci + 2 < NCHUNK)
        def _():
            fire_idx(ci + 2, nb)

        wait_gather(b)
        compute(b)
        fire_scatter(b)

    fire_idx(0, 0)
    fire_idx(1, 1)
    wait_idx(0)
    fire_gather(0)

    def loop_body(j, carry):
        for t in range(NBUF):
            chunk_step(j * NBUF + t, t)
        return carry

    lax.fori_loop(0, NCHUNK // NBUF, loop_body, 0)
    for ci in range(NCHUNK - NCHUNK % NBUF, NCHUNK):
        chunk_step(ci, ci % NBUF)
    wait_scatter((NCHUNK - 1) % NBUF)

    plsc.subcore_barrier()
    pltpu.sync_copy(acc.at[pl.ds(s * RPT, RPT)], out_hbm.at[c, pl.ds(s * RPT, RPT)])


def _sc_edge_entry(tsrc_hbm, tdst_hbm, src_hbm, dst_hbm, zero_hbm, out_hbm,
                   si0, si1, si2, di0, di1, di2, s0, s1, s2, d0, d1, d2,
                   acc, i0, i1, i2, g0, g1, g2, x0, x1, x2):
    _sc_edge_body(tsrc_hbm, tdst_hbm, src_hbm, dst_hbm, zero_hbm, out_hbm,
                  (si0, si1, si2), (di0, di1, di2),
                  (s0, s1, s2), (d0, d1, d2), acc,
                  (i0, i1, i2), (g0, g1, g2), (x0, x1, x2))


@functools.cache
def _sc_edge():
    return pl.kernel(
        _sc_edge_entry,
        out_type=jax.ShapeDtypeStruct((NC, NPAD, H), jnp.float32),
        mesh=plsc.VectorSubcoreMesh(core_axis_name="c", subcore_axis_name="s",
                                    num_cores=NC, num_subcores=NS),
        scratch_types=(
            [pltpu.VMEM((K,), jnp.int32) for _ in range(6)]
            + [pltpu.VMEM((K, 2 * D), jnp.float32) for _ in range(3)]
            + [pltpu.VMEM((K, H), jnp.float32) for _ in range(3)]
            + [pltpu.VMEM_SHARED((NPAD, H), jnp.float32)]
            + [pltpu.SemaphoreType.DMA for _ in range(9)]
        ),
    )


def _tc_finish(p_ref, wn_ref, bn_ref, wg_ref, bg_ref, wf_ref, bf_ref, out_ref):
    hn = p_ref[0, :N, :] + p_ref[1, :N, :]
    h = jnp.maximum(
        jnp.dot(hn, wn_ref[...], preferred_element_type=jnp.float32) + bn_ref[...],
        0.0)
    g = jnp.sum(h * wg_ref[...], axis=1, keepdims=True) + bg_ref[...]
    m = jnp.max(g)
    e = jnp.exp(g - m)
    ssum = jnp.sum(e)
    r = jnp.sum(e * h, axis=0, keepdims=True) / ssum
    out_ref[...] = jnp.dot(r, wf_ref[...], preferred_element_type=jnp.float32) + bf_ref[...]


def kernel(x, edge_index, W_edge, b_edge, W_node, b_node, W_gate, b_gate, W_fc, b_fc):
    tsrc, tdst = pl.pallas_call(
        _tc_prep,
        out_shape=[
            jax.ShapeDtypeStruct((N, 2 * D), jnp.float32),
            jax.ShapeDtypeStruct((N, H), jnp.float32),
        ],
    )(x, W_edge, b_edge.reshape(1, H))
    p = _sc_edge()(tsrc, tdst, edge_index[0], edge_index[1],
                   jnp.zeros((NPAD, H), jnp.float32))
    out = pl.pallas_call(
        _tc_finish,
        out_shape=jax.ShapeDtypeStruct((1, C), jnp.float32),
    )(p, W_node, b_node.reshape(1, H), W_gate.reshape(1, H),
      b_gate.reshape(1, 1), W_fc, b_fc.reshape(1, C))
    return out


# speedup vs baseline: 1.0191x; 1.0191x over previous
import functools

import jax
import jax.numpy as jnp
from jax import lax
from jax.experimental import pallas as pl
from jax.experimental.pallas import tpu as pltpu
from jax.experimental.pallas import tpu_sc as plsc

N = 10000
E = 320000
D = 128
H = 128
C = 10

NC = 2
NS = 16
NW = NC * NS
EW = E // NW
K = 40
NCHUNK = EW // K
NPAD = 10112
RPT = NPAD // NS
NBUF = 3


def _tc_prep(x_ref, we_ref, be_ref, tsrc_ref, tdst_ref):
    x = x_ref[...]
    a = jnp.dot(x, we_ref[:D, :], preferred_element_type=jnp.float32)
    b = jnp.dot(x, we_ref[D:, :], preferred_element_type=jnp.float32) + be_ref[...]
    tsrc_ref[:, :D] = x
    tsrc_ref[:, D:] = a
    tdst_ref[...] = b


def _sc_edge_body(tsrc_hbm, tdst_hbm, src_hbm, dst_hbm, zero_hbm, out_hbm,
                  sidx, didx, sbufs, dbufs, acc, isems, gsems, ssems):
    c = lax.axis_index("c")
    s = lax.axis_index("s")
    w = s * NC + c
    base_e = w * EW
    pltpu.sync_copy(zero_hbm.at[pl.ds(s * RPT, RPT)], acc.at[pl.ds(s * RPT, RPT)])
    plsc.subcore_barrier()

    def fire_idx(ci, b):
        pltpu.async_copy(src_hbm.at[pl.ds(base_e + ci * K, K)], sidx[b], isems[b])
        pltpu.async_copy(dst_hbm.at[pl.ds(base_e + ci * K, K)], didx[b], isems[b])

    def wait_idx(b):
        pltpu.make_async_copy(src_hbm.at[pl.ds(0, K)], sidx[b], isems[b]).wait()
        pltpu.make_async_copy(dst_hbm.at[pl.ds(0, K)], didx[b], isems[b]).wait()

    def fire_gather(b):
        pltpu.async_copy(tsrc_hbm.at[sidx[b]], sbufs[b], gsems[b])
        pltpu.async_copy(tdst_hbm.at[didx[b]], dbufs[b], gsems[b])

    def wait_gather(b):
        pltpu.make_async_copy(tsrc_hbm.at[sidx[b]], sbufs[b], gsems[b]).wait()
        pltpu.make_async_copy(tdst_hbm.at[didx[b]], dbufs[b], gsems[b]).wait()

    def fire_scatter(b):
        pltpu.async_copy(dbufs[b], acc.at[didx[b]], ssems[b], add=True)

    def wait_scatter(b):
        pltpu.make_async_copy(dbufs[b], acc.at[didx[b]], ssems[b]).wait()

    def compute(b):
        srows, drows = sbufs[b], dbufs[b]

        @plsc.parallel_loop(0, K, 1, unroll=4)
        def _(k):
            for j in range(H // 16):
                xv = srows[k, pl.ds(j * 16, 16)]
                av = srows[k, pl.ds(D + j * 16, 16)]
                bv = drows[k, pl.ds(j * 16, 16)]
                drows[k, pl.ds(j * 16, 16)] = xv * jnp.maximum(av + bv, 0.0)

    def chunk_step(ci, b):
        nb = (b + 2) % NBUF
        b1 = (b + 1) % NBUF

        @pl.when(ci + 1 < NCHUNK)
        def _():
            wait_idx(b1)
            fire_gather(b1)

        @pl.when(ci >= 1)
        def _():
            wait_scatter(nb)

        @pl.when(ci + 2 < NCHUNK)
        def _():
            fire_idx(ci + 2, nb)

        wait_gather(b)
        compute(b)
        fire_scatter(b)

    fire_idx(0, 0)
    fire_idx(1, 1)
    wait_idx(0)
    fire_gather(0)

    def loop_body(j, carry):
        for t in range(NBUF):
            chunk_step(j * NBUF + t, t)
        return carry

    lax.fori_loop(0, NCHUNK // NBUF, loop_body, 0)
    for ci in range(NCHUNK - NCHUNK % NBUF, NCHUNK):
        chunk_step(ci, ci % NBUF)
    wait_scatter((NCHUNK - 1) % NBUF)

    plsc.subcore_barrier()
    pltpu.sync_copy(acc.at[pl.ds(s * RPT, RPT)], out_hbm.at[c, pl.ds(s * RPT, RPT)])


def _sc_edge_entry(tsrc_hbm, tdst_hbm, src_hbm, dst_hbm, zero_hbm, out_hbm,
                   si0, si1, si2, di0, di1, di2, s0, s1, s2, d0, d1, d2,
                   acc, i0, i1, i2, g0, g1, g2, x0, x1, x2):
    _sc_edge_body(tsrc_hbm, tdst_hbm, src_hbm, dst_hbm, zero_hbm, out_hbm,
                  (si0, si1, si2), (di0, di1, di2),
                  (s0, s1, s2), (d0, d1, d2), acc,
                  (i0, i1, i2), (g0, g1, g2), (x0, x1, x2))


@functools.cache
def _sc_edge():
    return pl.kernel(
        _sc_edge_entry,
        out_type=jax.ShapeDtypeStruct((NC, NPAD, H), jnp.float32),
        mesh=plsc.VectorSubcoreMesh(core_axis_name="c", subcore_axis_name="s",
                                    num_cores=NC, num_subcores=NS),
        scratch_types=(
            [pltpu.VMEM((K,), jnp.int32) for _ in range(6)]
            + [pltpu.VMEM((K, 2 * D), jnp.float32) for _ in range(3)]
            + [pltpu.VMEM((K, H), jnp.float32) for _ in range(3)]
            + [pltpu.VMEM_SHARED((NPAD, H), jnp.float32)]
            + [pltpu.SemaphoreType.DMA for _ in range(9)]
        ),
    )


def _tc_finish(p_ref, wn_ref, bn_ref, wg_ref, bg_ref, wf_ref, bf_ref, out_ref):
    hn = p_ref[0, :N, :] + p_ref[1, :N, :]
    h = jnp.maximum(
        jnp.dot(hn, wn_ref[...], preferred_element_type=jnp.float32) + bn_ref[...],
        0.0)
    g = jnp.sum(h * wg_ref[...], axis=1, keepdims=True) + bg_ref[...]
    m = jnp.max(g)
    e = jnp.exp(g - m)
    ssum = jnp.sum(e)
    r = jnp.sum(e * h, axis=0, keepdims=True) / ssum
    out_ref[...] = jnp.dot(r, wf_ref[...], preferred_element_type=jnp.float32) + bf_ref[...]


def kernel(x, edge_index, W_edge, b_edge, W_node, b_node, W_gate, b_gate, W_fc, b_fc):
    tsrc, tdst = pl.pallas_call(
        _tc_prep,
        out_shape=[
            jax.ShapeDtypeStruct((N, 2 * D), jnp.float32),
            jax.ShapeDtypeStruct((N, H), jnp.float32),
        ],
    )(x, W_edge, b_edge.reshape(1, H))
    p = _sc_edge()(tsrc, tdst, edge_index[0], edge_index[1],
                   jnp.zeros((NPAD, H), jnp.float32))
    out = pl.pallas_call(
        _tc_finish,
        out_shape=jax.ShapeDtypeStruct((1, C), jnp.float32),
    )(p, W_node, b_node.reshape(1, H), W_gate.reshape(1, H),
      b_gate.reshape(1, 1), W_fc, b_fc.reshape(1, C))
    return out
